# baseline (speedup 1.0000x reference)
import jax
import jax.numpy as jnp
from jax import lax
from jax.experimental import pallas as pl

_BB = 32
_LP = 56
_KP = 384
_DM = 256


def _tables_body(ep_ref, ef_ref, ed_ref, wl_ref, bl_ref, wi_ref, bi_ref,
                 wf_ref, bf_ref, t_ref):
    wf = wf_ref[...]
    t_ref[0:256, :] = jnp.dot(ep_ref[...], wf[0:32, :],
                              preferred_element_type=jnp.float32)
    t_ref[256:320, :] = jnp.dot(ef_ref[...], wf[64:96, :],
                                preferred_element_type=jnp.float32)
    v_len = jnp.dot(wl_ref[...], wf[32:64, :],
                    preferred_element_type=jnp.float32)
    v_iat = jnp.dot(wi_ref[...], wf[96:128, :],
                    preferred_element_type=jnp.float32)
    td = jnp.dot(ed_ref[...], wf[128:136, :],
                 preferred_element_type=jnp.float32)
    c = (bf_ref[...]
         + jnp.dot(bl_ref[...], wf[32:64, :], preferred_element_type=jnp.float32)
         + jnp.dot(bi_ref[...], wf[96:128, :], preferred_element_type=jnp.float32)
         + td[0:1, :])
    d_vec = td[1:2, :] - td[0:1, :]
    tail = jnp.concatenate(
        [v_len, v_iat, d_vec, c, jnp.zeros((60, _DM), jnp.float32)], axis=0)
    t_ref[320:384, :] = tail


def _fwd_body(x_ref, t_ref, g_ref, b_ref, o_ref):
    n = _BB * _LP
    xb = x_ref[...].reshape(n, 5)
    pi = jnp.clip(xb[:, 0:1].astype(jnp.int32), 0, 255)
    fi = jnp.clip(xb[:, 2:3].astype(jnp.int32), 0, 63) + 256
    ln = xb[:, 1:2]
    it = xb[:, 3:4]
    dr = jnp.clip(xb[:, 4:5].astype(jnp.int32), 0, 1).astype(jnp.float32)
    iota = lax.broadcasted_iota(jnp.int32, (n, _KP), 1)
    onehot = jnp.where(jnp.logical_or(iota == pi, iota == fi),
                       jnp.float32(1.0), jnp.float32(0.0))
    t = t_ref[...]
    h = jnp.dot(onehot, t, preferred_element_type=jnp.float32)
    h = (h + ln * t[320:321, :] + it * t[321:322, :]
         + dr * t[322:323, :] + t[323:324, :])
    m = jnp.mean(h, axis=1, keepdims=True)
    d = h - m
    v = jnp.mean(d * d, axis=1, keepdims=True)
    out = d * lax.rsqrt(v + 1e-5) * g_ref[...] + b_ref[...]
    o_ref[...] = out.reshape(_BB, _LP, _DM)


def kernel(x, emb_proto, emb_flags, emb_dir, W_len, b_len, W_iat, b_iat,
           W_fusion, b_fusion, ln_gamma, ln_beta):
    B, L, _ = x.shape
    xp = jnp.pad(x, ((0, 0), (0, _LP - L), (0, 0)))
    wfT = W_fusion.T
    t = pl.pallas_call(
        _tables_body,
        out_shape=jax.ShapeDtypeStruct((_KP, _DM), jnp.float32),
    )(emb_proto, emb_flags, emb_dir,
      W_len.reshape(1, 32), b_len.reshape(1, 32),
      W_iat.reshape(1, 32), b_iat.reshape(1, 32),
      wfT, b_fusion.reshape(1, _DM))

    out3p = pl.pallas_call(
        _fwd_body,
        grid=(B // _BB,),
        in_specs=[
            pl.BlockSpec((_BB, _LP, 5), lambda i: (i, 0, 0)),
            pl.BlockSpec((_KP, _DM), lambda i: (0, 0)),
            pl.BlockSpec((1, _DM), lambda i: (0, 0)),
            pl.BlockSpec((1, _DM), lambda i: (0, 0)),
        ],
        out_specs=pl.BlockSpec((_BB, _LP, _DM), lambda i: (i, 0, 0)),
        out_shape=jax.ShapeDtypeStruct((B, _LP, _DM), jnp.float32),
    )(xp, t, ln_gamma.reshape(1, _DM), ln_beta.reshape(1, _DM))
    return out3p[:, :L, :]


# speedup vs baseline: 1.0425x; 1.0425x over previous
import jax
import jax.numpy as jnp
from jax import lax
from jax.experimental import pallas as pl

_BB = 128
_LB = 8
_KP = 384
_DM = 256


def _tables_body(ep_ref, ef_ref, ed_ref, wl_ref, bl_ref, wi_ref, bi_ref,
                 wf_ref, bf_ref, t_ref):
    wf = wf_ref[...]
    t_ref[0:256, :] = jnp.dot(ep_ref[...], wf[0:32, :],
                              preferred_element_type=jnp.float32)
    t_ref[256:320, :] = jnp.dot(ef_ref[...], wf[64:96, :],
                                preferred_element_type=jnp.float32)
    v_len = jnp.dot(wl_ref[...], wf[32:64, :],
                    preferred_element_type=jnp.float32)
    v_iat = jnp.dot(wi_ref[...], wf[96:128, :],
                    preferred_element_type=jnp.float32)
    td = jnp.dot(ed_ref[...], wf[128:136, :],
                 preferred_element_type=jnp.float32)
    c = (bf_ref[...]
         + jnp.dot(bl_ref[...], wf[32:64, :], preferred_element_type=jnp.float32)
         + jnp.dot(bi_ref[...], wf[96:128, :], preferred_element_type=jnp.float32)
         + td[0:1, :])
    d_vec = td[1:2, :] - td[0:1, :]
    tail = jnp.concatenate(
        [v_len, v_iat, d_vec, c, jnp.zeros((60, _DM), jnp.float32)], axis=0)
    t_ref[320:384, :] = tail


def _fwd_body(x_ref, t_ref, g_ref, b_ref, o_ref):
    n = _BB * _LB
    x3 = x_ref[...]
    pi3 = jnp.clip(x3[:, :, 0:1].astype(jnp.int32), 0, 255)
    fi3 = jnp.clip(x3[:, :, 2:3].astype(jnp.int32), 0, 63) + 256
    dr3 = jnp.clip(x3[:, :, 4:5].astype(jnp.int32), 0, 1).astype(jnp.float32)
    iota3 = lax.broadcasted_iota(jnp.int32, (_BB, _LB, _KP), 2)
    oh3 = jnp.where(jnp.logical_or(iota3 == pi3, iota3 == fi3),
                    jnp.float32(1.0), jnp.float32(0.0))
    t = t_ref[...]
    h2 = jnp.dot(oh3.reshape(n, _KP), t,
                 preferred_element_type=jnp.float32)
    h3 = h2.reshape(_BB, _LB, _DM)
    h3 = (h3
          + x3[:, :, 1:2] * t[320:321, :].reshape(1, 1, _DM)
          + x3[:, :, 3:4] * t[321:322, :].reshape(1, 1, _DM)
          + dr3 * t[322:323, :].reshape(1, 1, _DM)
          + t[323:324, :].reshape(1, 1, _DM))
    m = jnp.mean(h3, axis=2, keepdims=True)
    d = h3 - m
    v = jnp.mean(d * d, axis=2, keepdims=True)
    o_ref[...] = d * lax.rsqrt(v + 1e-5) * g_ref[...] + b_ref[...]


def kernel(x, emb_proto, emb_flags, emb_dir, W_len, b_len, W_iat, b_iat,
           W_fusion, b_fusion, ln_gamma, ln_beta):
    B, L, _ = x.shape
    wfT = W_fusion.T
    t = pl.pallas_call(
        _tables_body,
        out_shape=jax.ShapeDtypeStruct((_KP, _DM), jnp.float32),
    )(emb_proto, emb_flags, emb_dir,
      W_len.reshape(1, 32), b_len.reshape(1, 32),
      W_iat.reshape(1, 32), b_iat.reshape(1, 32),
      wfT, b_fusion.reshape(1, _DM))

    nl = (L + _LB - 1) // _LB
    out3 = pl.pallas_call(
        _fwd_body,
        grid=(B // _BB, nl),
        in_specs=[
            pl.BlockSpec((_BB, _LB, 5), lambda i, j: (i, j, 0)),
            pl.BlockSpec((_KP, _DM), lambda i, j: (0, 0)),
            pl.BlockSpec((1, 1, _DM), lambda i, j: (0, 0, 0)),
            pl.BlockSpec((1, 1, _DM), lambda i, j: (0, 0, 0)),
        ],
        out_specs=pl.BlockSpec((_BB, _LB, _DM), lambda i, j: (i, j, 0)),
        out_shape=jax.ShapeDtypeStruct((B, L, _DM), jnp.float32),
    )(x, t, ln_gamma.reshape(1, 1, _DM), ln_beta.reshape(1, 1, _DM))
    return out3
